# baseline (speedup 1.0000x reference)
import functools

import jax
import jax.numpy as jnp
from jax import lax
from jax.experimental import pallas as pl
from jax.experimental.pallas import tpu as pltpu
from jax.experimental.pallas import tpu_sc as plsc

BATCH = 16384
EMBED_DIM = 32
NUM_ROWS = 1000000

NUM_CORES = 2
NUM_SUBCORES = 16
NW = NUM_CORES * NUM_SUBCORES
BPW = BATCH // NW
VL = 16

NSTR = BPW * EMBED_DIM // 128
FIRE = 8

MLP_BLK = 2048


def _make_sc_gather():
    mesh = plsc.VectorSubcoreMesh(core_axis_name="c", subcore_axis_name="s")

    @functools.partial(
        pl.kernel,
        mesh=mesh,
        compiler_params=pltpu.CompilerParams(use_tc_tiling_on_sc=False),
        out_type=[
            jax.ShapeDtypeStruct((EMBED_DIM, BATCH), jnp.float32),
            jax.ShapeDtypeStruct((EMBED_DIM, BATCH), jnp.float32),
        ],
        scratch_types=[
            pltpu.VMEM((BPW,), jnp.int32),
            pltpu.VMEM((BPW,), jnp.int32),
            pltpu.VMEM((NSTR, 128), jnp.int32),
            pltpu.VMEM((NSTR, 128), jnp.int32),
            pltpu.VMEM((EMBED_DIM, BPW), jnp.float32),
            pltpu.VMEM((EMBED_DIM, BPW), jnp.float32),
            pltpu.SemaphoreType.DMA,
        ],
    )
    def gather(uid_hbm, iid_hbm, uflat_hbm, iflat_hbm, ue_out, ie_out,
               uids_v, iids_v, uidx_v, iidx_v, uvals_v, ivals_v, sem):
        wid = lax.axis_index("s") * NUM_CORES + lax.axis_index("c")
        base = wid * BPW
        pltpu.sync_copy(uid_hbm.at[wid], uids_v)
        pltpu.sync_copy(iid_hbm.at[wid], iids_v)

        nblk = BPW // 128
        ngrp = 128 // VL

        def build(g, carry):
            p = g // ngrp
            v = g % ngrp
            rvu = uids_v[pl.ds(p * 128 + v * VL, VL)]
            rvi = iids_v[pl.ds(p * 128 + v * VL, VL)]
            def per_c(c, carry2):
                m = c * nblk + p
                uidx_v[m, pl.ds(v * VL, VL)] = rvu + c * NUM_ROWS
                iidx_v[m, pl.ds(v * VL, VL)] = rvi + c * NUM_ROWS
                return carry2
            lax.fori_loop(0, EMBED_DIM, per_c, 0)
            return carry

        lax.fori_loop(0, BPW // VL, build, 0)

        def fire_group(t, carry):
            copies = []
            for j in range(FIRE):
                m = t * FIRE + j
                c = m // nblk
                p = m % nblk
                copies.append(pltpu.async_copy(
                    uflat_hbm.at[uidx_v.at[m]],
                    uvals_v.at[c, pl.ds(p * 128, 128)], sem))
                copies.append(pltpu.async_copy(
                    iflat_hbm.at[iidx_v.at[m]],
                    ivals_v.at[c, pl.ds(p * 128, 128)], sem))
            for cp in copies:
                cp.wait()
            return carry

        lax.fori_loop(0, NSTR // FIRE, fire_group, 0)

        pltpu.sync_copy(uvals_v, ue_out.at[:, pl.ds(base, BPW)])
        pltpu.sync_copy(ivals_v, ie_out.at[:, pl.ds(base, BPW)])

    return gather


_sc_gather = _make_sc_gather()


def _mlp_body(ue_t, ie_t, a0u, a0i, b0, a1, b1, a2, b2, ao, bo, out):
    h = jnp.dot(a0u[...], ue_t[...], preferred_element_type=jnp.float32)
    h = h + jnp.dot(a0i[...], ie_t[...], preferred_element_type=jnp.float32)
    h = jnp.maximum(h + b0[...], 0.0)
    h = jnp.maximum(jnp.dot(a1[...], h, preferred_element_type=jnp.float32) + b1[...], 0.0)
    h = jnp.maximum(jnp.dot(a2[...], h, preferred_element_type=jnp.float32) + b2[...], 0.0)
    out[...] = jnp.dot(ao[...], h, preferred_element_type=jnp.float32) + bo[...]


def _tc_mlp(ue_t, ie_t, A0u, A0i, b0, A1, b1, A2, b2, Ao, bo):
    grid = (BATCH // MLP_BLK,)
    full = lambda shape: pl.BlockSpec(shape, lambda i: (0,) * len(shape))
    return pl.pallas_call(
        _mlp_body,
        grid=grid,
        in_specs=[
            pl.BlockSpec((EMBED_DIM, MLP_BLK), lambda i: (0, i)),
            pl.BlockSpec((EMBED_DIM, MLP_BLK), lambda i: (0, i)),
            full(A0u.shape), full(A0i.shape), full(b0.shape),
            full(A1.shape), full(b1.shape),
            full(A2.shape), full(b2.shape),
            full(Ao.shape), full(bo.shape),
        ],
        out_specs=pl.BlockSpec((1, MLP_BLK), lambda i: (0, i)),
        out_shape=jax.ShapeDtypeStruct((1, BATCH), jnp.float32),
    )(ue_t, ie_t, A0u, A0i, b0, A1, b1, A2, b2, Ao, bo)


def kernel(user_ids, item_ids, user_emb, item_emb, W0, b0, W1, b1, W2, b2, Wo, bo):
    uid = user_ids.reshape(NW, BPW)
    iid = item_ids.reshape(NW, BPW)
    ue_t, ie_t = _sc_gather(uid, iid,
                            user_emb.T.reshape(-1),
                            item_emb.T.reshape(-1))
    out = _tc_mlp(
        ue_t, ie_t,
        W0[:EMBED_DIM].T, W0[EMBED_DIM:].T, b0.reshape(-1, 1),
        W1.T, b1.reshape(-1, 1), W2.T, b2.reshape(-1, 1),
        Wo.T, bo.reshape(1, 1),
    )
    return out.reshape(BATCH)


# speedup vs baseline: 19.2071x; 19.2071x over previous
import functools

import jax
import jax.numpy as jnp
from jax import lax
from jax.experimental import pallas as pl
from jax.experimental.pallas import tpu as pltpu
from jax.experimental.pallas import tpu_sc as plsc

BATCH = 16384
EMBED_DIM = 32
NUM_ROWS = 1000000

NUM_CORES = 2
NUM_SUBCORES = 16
NW = NUM_CORES * NUM_SUBCORES
BPW = BATCH // NW
VL = 16

NSTR = BPW * EMBED_DIM // 128
FIRE = 8

MLP_BLK = 2048


RP_LANES = 16384
TILE_PITCH = 7936
PITCH = TILE_PITCH * 128


def _repack_body(x_ref, o_ref):
    o_ref[...] = x_ref[...].reshape(EMBED_DIM, RP_LANES // 128, 128)


def _repack(t):
    grid = (pl.cdiv(NUM_ROWS, RP_LANES),)
    return pl.pallas_call(
        _repack_body,
        grid=grid,
        in_specs=[pl.BlockSpec((EMBED_DIM, RP_LANES), lambda i: (0, i))],
        out_specs=pl.BlockSpec((EMBED_DIM, RP_LANES // 128, 128), lambda i: (0, i, 0)),
        out_shape=jax.ShapeDtypeStruct((EMBED_DIM, TILE_PITCH, 128), jnp.float32),
    )(t)


def _make_sc_gather():
    mesh = plsc.VectorSubcoreMesh(core_axis_name="c", subcore_axis_name="s")

    @functools.partial(
        pl.kernel,
        mesh=mesh,
        compiler_params=pltpu.CompilerParams(use_tc_tiling_on_sc=False),
        out_type=[
            jax.ShapeDtypeStruct((EMBED_DIM, BATCH), jnp.float32),
            jax.ShapeDtypeStruct((EMBED_DIM, BATCH), jnp.float32),
        ],
        scratch_types=[
            pltpu.VMEM((BPW // 128, 128), jnp.int32),
            pltpu.VMEM((BPW // 128, 128), jnp.int32),
            pltpu.VMEM((EMBED_DIM, BPW), jnp.float32),
            pltpu.VMEM((EMBED_DIM, BPW), jnp.float32),
            pltpu.SemaphoreType.DMA,
        ],
    )
    def gather(uid_hbm, iid_hbm, ut_hbm, it_hbm, ue_out, ie_out,
               uids_v, iids_v, uvals_v, ivals_v, sem):
        wid = lax.axis_index("s") * NUM_CORES + lax.axis_index("c")
        base = wid * BPW
        nblk = BPW // 128
        pltpu.sync_copy(uid_hbm.at[wid], uids_v)
        pltpu.sync_copy(iid_hbm.at[wid], iids_v)

        def fire_group(t, carry):
            copies = []
            for j in range(FIRE):
                m = t * FIRE + j
                c = m // nblk
                p = m % nblk
                copies.append(pltpu.async_copy(
                    ut_hbm.at[c].at[uids_v.at[p]],
                    uvals_v.at[c].at[pl.ds(p * 128, 128)], sem))
                copies.append(pltpu.async_copy(
                    it_hbm.at[c].at[iids_v.at[p]],
                    ivals_v.at[c].at[pl.ds(p * 128, 128)], sem))
            for cp in copies:
                cp.wait()
            return carry

        lax.fori_loop(0, NSTR // FIRE, fire_group, 0)

        pltpu.sync_copy(uvals_v, ue_out.at[:, pl.ds(base, BPW)])
        pltpu.sync_copy(ivals_v, ie_out.at[:, pl.ds(base, BPW)])

    return gather


_sc_gather = _make_sc_gather()


def _mlp_body(ue_t, ie_t, a0u, a0i, b0, a1, b1, a2, b2, ao, bo, out):
    h = jnp.dot(a0u[...], ue_t[...], preferred_element_type=jnp.float32)
    h = h + jnp.dot(a0i[...], ie_t[...], preferred_element_type=jnp.float32)
    h = jnp.maximum(h + b0[...], 0.0)
    h = jnp.maximum(jnp.dot(a1[...], h, preferred_element_type=jnp.float32) + b1[...], 0.0)
    h = jnp.maximum(jnp.dot(a2[...], h, preferred_element_type=jnp.float32) + b2[...], 0.0)
    out[...] = jnp.dot(ao[...], h, preferred_element_type=jnp.float32) + bo[...]


def _tc_mlp(ue_t, ie_t, A0u, A0i, b0, A1, b1, A2, b2, Ao, bo):
    grid = (BATCH // MLP_BLK,)
    full = lambda shape: pl.BlockSpec(shape, lambda i: (0,) * len(shape))
    return pl.pallas_call(
        _mlp_body,
        grid=grid,
        in_specs=[
            pl.BlockSpec((EMBED_DIM, MLP_BLK), lambda i: (0, i)),
            pl.BlockSpec((EMBED_DIM, MLP_BLK), lambda i: (0, i)),
            full(A0u.shape), full(A0i.shape), full(b0.shape),
            full(A1.shape), full(b1.shape),
            full(A2.shape), full(b2.shape),
            full(Ao.shape), full(bo.shape),
        ],
        out_specs=pl.BlockSpec((1, MLP_BLK), lambda i: (0, i)),
        out_shape=jax.ShapeDtypeStruct((1, BATCH), jnp.float32),
    )(ue_t, ie_t, A0u, A0i, b0, A1, b1, A2, b2, Ao, bo)


def kernel(user_ids, item_ids, user_emb, item_emb, W0, b0, W1, b1, W2, b2, Wo, bo):
    uid = user_ids.reshape(NW, BPW // 128, 128)
    iid = item_ids.reshape(NW, BPW // 128, 128)
    u3 = _repack(user_emb.T)
    i3 = _repack(item_emb.T)
    ue_t, ie_t = _sc_gather(uid, iid,
                            u3.reshape(EMBED_DIM, PITCH),
                            i3.reshape(EMBED_DIM, PITCH))
    out = _tc_mlp(
        ue_t, ie_t,
        W0[:EMBED_DIM].T, W0[EMBED_DIM:].T, b0.reshape(-1, 1),
        W1.T, b1.reshape(-1, 1), W2.T, b2.reshape(-1, 1),
        Wo.T, bo.reshape(1, 1),
    )
    return out.reshape(BATCH)


# speedup vs baseline: 19.6047x; 1.0207x over previous
import functools

import jax
import jax.numpy as jnp
from jax import lax
from jax.experimental import pallas as pl
from jax.experimental.pallas import tpu as pltpu
from jax.experimental.pallas import tpu_sc as plsc

BATCH = 16384
EMBED_DIM = 32
NUM_ROWS = 1000000

NUM_CORES = 2
NUM_SUBCORES = 16
NW = NUM_CORES * NUM_SUBCORES
BPW = BATCH // NW
VL = 16

NSTR = BPW * EMBED_DIM // 128
FIRE = 16

MLP_BLK = 2048


RP_LANES = 16384
TILE_PITCH = 7936
PITCH = TILE_PITCH * 128


def _repack_body(x_ref, o_ref):
    o_ref[...] = x_ref[...].reshape(EMBED_DIM, RP_LANES // 128, 128)


def _repack(t):
    grid = (pl.cdiv(NUM_ROWS, RP_LANES),)
    return pl.pallas_call(
        _repack_body,
        grid=grid,
        in_specs=[pl.BlockSpec((EMBED_DIM, RP_LANES), lambda i: (0, i))],
        out_specs=pl.BlockSpec((EMBED_DIM, RP_LANES // 128, 128), lambda i: (0, i, 0)),
        out_shape=jax.ShapeDtypeStruct((EMBED_DIM, TILE_PITCH, 128), jnp.float32),
    )(t)


def _make_sc_gather():
    mesh = plsc.VectorSubcoreMesh(core_axis_name="c", subcore_axis_name="s")

    @functools.partial(
        pl.kernel,
        mesh=mesh,
        compiler_params=pltpu.CompilerParams(use_tc_tiling_on_sc=False),
        out_type=[
            jax.ShapeDtypeStruct((EMBED_DIM, BATCH), jnp.float32),
            jax.ShapeDtypeStruct((EMBED_DIM, BATCH), jnp.float32),
        ],
        scratch_types=[
            pltpu.VMEM((BPW // 128, 128), jnp.int32),
            pltpu.VMEM((BPW // 128, 128), jnp.int32),
            pltpu.VMEM((EMBED_DIM, BPW), jnp.float32),
            pltpu.VMEM((EMBED_DIM, BPW), jnp.float32),
            pltpu.SemaphoreType.DMA,
        ],
    )
    def gather(uid_hbm, iid_hbm, ut_hbm, it_hbm, ue_out, ie_out,
               uids_v, iids_v, uvals_v, ivals_v, sem):
        wid = lax.axis_index("s") * NUM_CORES + lax.axis_index("c")
        base = wid * BPW
        nblk = BPW // 128
        pltpu.sync_copy(uid_hbm.at[wid], uids_v)
        pltpu.sync_copy(iid_hbm.at[wid], iids_v)

        def fire_group(t, carry):
            copies = []
            for j in range(FIRE):
                m = t * FIRE + j
                c = m // nblk
                p = m % nblk
                copies.append(pltpu.async_copy(
                    ut_hbm.at[c].at[uids_v.at[p]],
                    uvals_v.at[c].at[pl.ds(p * 128, 128)], sem))
                copies.append(pltpu.async_copy(
                    it_hbm.at[c].at[iids_v.at[p]],
                    ivals_v.at[c].at[pl.ds(p * 128, 128)], sem))
            for cp in copies:
                cp.wait()
            return carry

        lax.fori_loop(0, NSTR // FIRE, fire_group, 0)

        pltpu.sync_copy(uvals_v, ue_out.at[:, pl.ds(base, BPW)])
        pltpu.sync_copy(ivals_v, ie_out.at[:, pl.ds(base, BPW)])

    return gather


_sc_gather = _make_sc_gather()


def _mlp_body(ue_t, ie_t, a0u, a0i, b0, a1, b1, a2, b2, ao, bo, out):
    h = jnp.dot(a0u[...], ue_t[...], preferred_element_type=jnp.float32)
    h = h + jnp.dot(a0i[...], ie_t[...], preferred_element_type=jnp.float32)
    h = jnp.maximum(h + b0[...], 0.0)
    h = jnp.maximum(jnp.dot(a1[...], h, preferred_element_type=jnp.float32) + b1[...], 0.0)
    h = jnp.maximum(jnp.dot(a2[...], h, preferred_element_type=jnp.float32) + b2[...], 0.0)
    out[...] = jnp.dot(ao[...], h, preferred_element_type=jnp.float32) + bo[...]


def _tc_mlp(ue_t, ie_t, A0u, A0i, b0, A1, b1, A2, b2, Ao, bo):
    grid = (BATCH // MLP_BLK,)
    full = lambda shape: pl.BlockSpec(shape, lambda i: (0,) * len(shape))
    return pl.pallas_call(
        _mlp_body,
        grid=grid,
        in_specs=[
            pl.BlockSpec((EMBED_DIM, MLP_BLK), lambda i: (0, i)),
            pl.BlockSpec((EMBED_DIM, MLP_BLK), lambda i: (0, i)),
            full(A0u.shape), full(A0i.shape), full(b0.shape),
            full(A1.shape), full(b1.shape),
            full(A2.shape), full(b2.shape),
            full(Ao.shape), full(bo.shape),
        ],
        out_specs=pl.BlockSpec((1, MLP_BLK), lambda i: (0, i)),
        out_shape=jax.ShapeDtypeStruct((1, BATCH), jnp.float32),
    )(ue_t, ie_t, A0u, A0i, b0, A1, b1, A2, b2, Ao, bo)


def kernel(user_ids, item_ids, user_emb, item_emb, W0, b0, W1, b1, W2, b2, Wo, bo):
    uid = user_ids.reshape(NW, BPW // 128, 128)
    iid = item_ids.reshape(NW, BPW // 128, 128)
    u3 = _repack(user_emb.T)
    i3 = _repack(item_emb.T)
    ue_t, ie_t = _sc_gather(uid, iid,
                            u3.reshape(EMBED_DIM, PITCH),
                            i3.reshape(EMBED_DIM, PITCH))
    out = _tc_mlp(
        ue_t, ie_t,
        W0[:EMBED_DIM].T, W0[EMBED_DIM:].T, b0.reshape(-1, 1),
        W1.T, b1.reshape(-1, 1), W2.T, b2.reshape(-1, 1),
        Wo.T, bo.reshape(1, 1),
    )
    return out.reshape(BATCH)


# speedup vs baseline: 20.6375x; 1.0527x over previous
import functools

import jax
import jax.numpy as jnp
from jax import lax
from jax.experimental import pallas as pl
from jax.experimental.pallas import tpu as pltpu
from jax.experimental.pallas import tpu_sc as plsc

BATCH = 16384
EMBED_DIM = 32
NUM_ROWS = 1000000

NUM_CORES = 2
NUM_SUBCORES = 16
NW = NUM_CORES * NUM_SUBCORES
BPW = BATCH // NW
VL = 16

NSTR = BPW * EMBED_DIM // 128
FIRE = 16

MLP_BLK = 2048


RP_LANES = 16384
TILE_PITCH = 7936
PITCH = TILE_PITCH * 128


def _repack_body(x_ref, o_ref):
    o_ref[...] = x_ref[...].reshape(EMBED_DIM, RP_LANES // 128, 128)


def _repack(t):
    grid = (pl.cdiv(NUM_ROWS, RP_LANES),)
    return pl.pallas_call(
        _repack_body,
        grid=grid,
        in_specs=[pl.BlockSpec((EMBED_DIM, RP_LANES), lambda i: (0, i))],
        out_specs=pl.BlockSpec((EMBED_DIM, RP_LANES // 128, 128), lambda i: (0, i, 0)),
        out_shape=jax.ShapeDtypeStruct((EMBED_DIM, TILE_PITCH, 128), jnp.float32),
    )(t)


def _make_sc_gather():
    mesh = plsc.VectorSubcoreMesh(core_axis_name="c", subcore_axis_name="s")

    @functools.partial(
        pl.kernel,
        mesh=mesh,
        compiler_params=pltpu.CompilerParams(use_tc_tiling_on_sc=False),
        out_type=jax.ShapeDtypeStruct((EMBED_DIM, BATCH), jnp.float32),
        scratch_types=[
            pltpu.VMEM((BPW // 128, 128), jnp.int32),
            pltpu.VMEM((EMBED_DIM, BPW), jnp.float32),
            pltpu.SemaphoreType.DMA,
        ],
    )
    def gather(id_hbm, tab_hbm, out_hbm, ids_v, vals_v, sem):
        wid = lax.axis_index("s") * NUM_CORES + lax.axis_index("c")
        base = wid * BPW
        nblk = BPW // 128
        pltpu.sync_copy(id_hbm.at[wid], ids_v)

        def fire_group(t, carry):
            copies = []
            for j in range(FIRE):
                m = t * FIRE + j
                c = m // nblk
                p = m % nblk
                copies.append(pltpu.async_copy(
                    tab_hbm.at[c].at[ids_v.at[p]],
                    vals_v.at[c].at[pl.ds(p * 128, 128)], sem))
            for cp in copies:
                cp.wait()
            return carry

        lax.fori_loop(0, NSTR // FIRE, fire_group, 0)
        pltpu.sync_copy(vals_v, out_hbm.at[:, pl.ds(base, BPW)])

    return gather


_sc_gather = _make_sc_gather()


def _mlp_body(ue_t, ie_t, a0u, a0i, b0, a1, b1, a2, b2, ao, bo, out):
    h = jnp.dot(a0u[...], ue_t[...], preferred_element_type=jnp.float32)
    h = h + jnp.dot(a0i[...], ie_t[...], preferred_element_type=jnp.float32)
    h = jnp.maximum(h + b0[...], 0.0)
    h = jnp.maximum(jnp.dot(a1[...], h, preferred_element_type=jnp.float32) + b1[...], 0.0)
    h = jnp.maximum(jnp.dot(a2[...], h, preferred_element_type=jnp.float32) + b2[...], 0.0)
    out[...] = jnp.dot(ao[...], h, preferred_element_type=jnp.float32) + bo[...]


def _tc_mlp(ue_t, ie_t, A0u, A0i, b0, A1, b1, A2, b2, Ao, bo):
    grid = (BATCH // MLP_BLK,)
    full = lambda shape: pl.BlockSpec(shape, lambda i: (0,) * len(shape))
    return pl.pallas_call(
        _mlp_body,
        grid=grid,
        in_specs=[
            pl.BlockSpec((EMBED_DIM, MLP_BLK), lambda i: (0, i)),
            pl.BlockSpec((EMBED_DIM, MLP_BLK), lambda i: (0, i)),
            full(A0u.shape), full(A0i.shape), full(b0.shape),
            full(A1.shape), full(b1.shape),
            full(A2.shape), full(b2.shape),
            full(Ao.shape), full(bo.shape),
        ],
        out_specs=pl.BlockSpec((1, MLP_BLK), lambda i: (0, i)),
        out_shape=jax.ShapeDtypeStruct((1, BATCH), jnp.float32),
    )(ue_t, ie_t, A0u, A0i, b0, A1, b1, A2, b2, Ao, bo)


def kernel(user_ids, item_ids, user_emb, item_emb, W0, b0, W1, b1, W2, b2, Wo, bo):
    uid = user_ids.reshape(NW, BPW // 128, 128)
    iid = item_ids.reshape(NW, BPW // 128, 128)
    u3 = _repack(user_emb.T)
    ue_t = _sc_gather(uid, u3.reshape(EMBED_DIM, PITCH))
    i3 = _repack(item_emb.T)
    ie_t = _sc_gather(iid, i3.reshape(EMBED_DIM, PITCH))
    out = _tc_mlp(
        ue_t, ie_t,
        W0[:EMBED_DIM].T, W0[EMBED_DIM:].T, b0.reshape(-1, 1),
        W1.T, b1.reshape(-1, 1), W2.T, b2.reshape(-1, 1),
        Wo.T, bo.reshape(1, 1),
    )
    return out.reshape(BATCH)
